# baseline (speedup 1.0000x reference)
import jax
import jax.numpy as jnp
from jax import lax
from jax.experimental import pallas as pl
from jax.experimental.pallas import tpu as pltpu
from jax.experimental.pallas import tpu_sc as plsc

B, L, H = 16, 4096, 256
NC, NS = 2, 16
NW = NC * NS
WPB = NW // B
TOK_PER_W = L // WPB
TT = 128
NT = TOK_PER_W // TT
RC = 112
HV = H // 16


def _sget(ref, idx):
    return ref[pl.ds(idx, 16)][0]


def _lower_bound(seg_v, t):
    def step(_, lh):
        lo, hi = lh
        mid = (lo + hi) // 2
        v = _sget(seg_v, mid)
        active = lo < hi
        lt = (v < t) & active
        ge = jnp.logical_not(v < t) & active
        return jnp.where(lt, mid + 1, lo), jnp.where(ge, mid, hi)
    lo, _ = lax.fori_loop(0, 12, step, (jnp.int32(0), jnp.int32(L)))
    return lo


def _body(x_hbm, seg_hbm, out_hbm, seg_v, in0, in1, ob0, ob1, zshared,
          sem_i0, sem_i1, sem_o0, sem_o1, sem_z):
    c = lax.axis_index("c")
    s = lax.axis_index("s")
    wid = s * NC + c
    b = wid // WPB
    t_base = (wid % WPB) * TOK_PER_W

    inbufs = (in0, in1)
    outbufs = (ob0, ob1)
    sems_i = (sem_i0, sem_i1)
    sems_o = (sem_o0, sem_o1)
    zv = jnp.zeros((16,), jnp.float32)

    @pl.when(s == 0)
    def _():
        def zb(t, carry):
            for h in range(HV):
                ob0[t, pl.ds(16 * h, 16)] = zv
            return carry
        lax.fori_loop(0, TT, zb, 0)
        pltpu.sync_copy(ob0, zshared)

    plsc.subcore_barrier()

    pltpu.sync_copy(seg_hbm.at[pl.ds(b * L, L)], seg_v.at[pl.ds(0, L)])
    seg_v[pl.ds(L, 16)] = jnp.full((16,), L, jnp.int32)
    r_start = _lower_bound(seg_v, t_base)
    r_end = _lower_bound(seg_v, t_base + TOK_PER_W)

    a0w = lax.bitwise_and(r_start, -8)
    nchw = (r_end - a0w + RC - 1) // RC

    def in_dma(ci, buf, sem):
        start = a0w + ci * RC
        d = jnp.minimum(start, L - RC)
        d = pl.multiple_of(d, 8)
        return pltpu.make_async_copy(x_hbm.at[b, pl.ds(d, RC), :], buf, sem)

    def out_dma(k, buf, sem):
        t_lo = pl.multiple_of(t_base + k * TT, TT)
        return pltpu.make_async_copy(buf, out_hbm.at[b, pl.ds(t_lo, TT), :], sem)

    @pl.when(nchw > 0)
    def _():
        in_dma(0, inbufs[0], sems_i[0]).start()

    def run_tile(kq, q, r0, lw):
        k = 2 * kq + q
        t_lo = t_base + k * TT
        r1 = _lower_bound(seg_v, t_lo + TT)
        outbuf = outbufs[q]

        @pl.when(kq >= 1)
        def _():
            out_dma(k - 2, outbuf, sems_o[q]).wait()

        zdma = pltpu.make_async_copy(zshared, outbuf, sem_z)
        zdma.start()
        zdma.wait()

        ci_lo = (r0 - a0w) // RC
        ci_hi = (r1 - a0w + RC - 1) // RC

        def chunk_pair(cj, carry):
            lw = carry[0]
            for p in (0, 1):
                ci = 2 * cj + p
                start = a0w + ci * RC
                d = jnp.minimum(start, L - RC)
                lo_i = jnp.maximum(r0, start)
                cnt = jnp.clip(jnp.minimum(r1, start + RC) - lo_i, 0, RC)
                fresh = (ci > lw) & (cnt > 0)

                @pl.when(fresh)
                def _():
                    in_dma(ci, inbufs[p], sems_i[p]).wait()

                    @pl.when(ci + 1 < nchw)
                    def _():
                        in_dma(ci + 1, inbufs[1 - p], sems_i[1 - p]).start()

                lw = jnp.where(fresh, ci, lw)
                inbuf = inbufs[p]

                def do_row(pr, rc):
                    prev_s = rc[0]
                    acc = rc[1:]
                    sgid = _sget(seg_v, pr)
                    lr = pr - d
                    same = sgid == prev_s
                    new_acc = []
                    for h in range(HV):
                        xv = inbuf[lr, pl.ds(16 * h, 16)]
                        new_acc.append(xv + jnp.where(same, acc[h], zv))
                    tl = sgid - t_lo
                    for h in range(HV):
                        outbuf[tl, pl.ds(16 * h, 16)] = new_acc[h]
                    return (sgid,) + tuple(new_acc)

                def row_body(i, rc):
                    return do_row(lo_i + i, rc)

                rc = lax.fori_loop(0, cnt, row_body, carry[1:])
                carry = (lw,) + rc
            return carry

        init = (lw, t_lo - 1) + tuple(zv for _ in range(HV))
        fin = lax.fori_loop(ci_lo // 2, (ci_hi + 1) // 2, chunk_pair, init)
        out_dma(k, outbuf, sems_o[q]).start()
        return r1, fin[0]

    def tile_pair(kq, carry):
        r0, lw = carry
        r0, lw = run_tile(kq, 0, r0, lw)
        r0, lw = run_tile(kq, 1, r0, lw)
        return r0, lw

    lax.fori_loop(0, NT // 2, tile_pair, (r_start, jnp.int32(-1)))
    out_dma(NT - 2, outbufs[0], sems_o[0]).wait()
    out_dma(NT - 1, outbufs[1], sems_o[1]).wait()


@jax.jit
def kernel(sequence_output, wp_segment_ids):
    seg = wp_segment_ids.astype(jnp.int32).reshape(B * L)
    run = pl.kernel(
        _body,
        out_type=jax.ShapeDtypeStruct((B, L, H), jnp.float32),
        mesh=plsc.VectorSubcoreMesh(core_axis_name="c", subcore_axis_name="s"),
        compiler_params=pltpu.CompilerParams(use_tc_tiling_on_sc=True),
        scratch_types=[
            pltpu.VMEM((L + 16,), jnp.int32),
            pltpu.VMEM((RC, H), jnp.float32),
            pltpu.VMEM((RC, H), jnp.float32),
            pltpu.VMEM((TT, H), jnp.float32),
            pltpu.VMEM((TT, H), jnp.float32),
            pltpu.VMEM_SHARED((TT, H), jnp.float32),
            pltpu.SemaphoreType.DMA,
            pltpu.SemaphoreType.DMA,
            pltpu.SemaphoreType.DMA,
            pltpu.SemaphoreType.DMA,
            pltpu.SemaphoreType.DMA,
        ],
    )
    return run(sequence_output, seg)


# speedup vs baseline: 1.0330x; 1.0330x over previous
import jax
import jax.numpy as jnp
from jax import lax
from jax.experimental import pallas as pl
from jax.experimental.pallas import tpu as pltpu
from jax.experimental.pallas import tpu_sc as plsc

B, L, H = 16, 4096, 256
NC, NS = 2, 16
NW = NC * NS
WPB = NW // B
TOK_PER_W = L // WPB
TT = 128
NT = TOK_PER_W // TT
RC = 112
HV = H // 16


def _sget(ref, idx):
    return ref[pl.ds(idx, 16)][0]


def _lower_bound(seg_v, t):
    def step(_, lh):
        lo, hi = lh
        mid = (lo + hi) // 2
        v = _sget(seg_v, mid)
        active = lo < hi
        lt = (v < t) & active
        ge = jnp.logical_not(v < t) & active
        return jnp.where(lt, mid + 1, lo), jnp.where(ge, mid, hi)
    lo, _ = lax.fori_loop(0, 12, step, (jnp.int32(0), jnp.int32(L)))
    return lo


def _body(x_hbm, seg_hbm, out_hbm, seg_v, in0, in1, ob0, ob1, zshared,
          sem_i0, sem_i1, sem_o0, sem_o1, sem_z):
    c = lax.axis_index("c")
    s = lax.axis_index("s")
    wid = s * NC + c
    b = wid // WPB
    t_base = (wid % WPB) * TOK_PER_W

    inbufs = (in0, in1)
    outbufs = (ob0, ob1)
    sems_i = (sem_i0, sem_i1)
    sems_o = (sem_o0, sem_o1)
    zv = jnp.zeros((16,), jnp.float32)

    @pl.when(s == 0)
    def _():
        def zb(t, carry):
            for h in range(HV):
                ob0[t, pl.ds(16 * h, 16)] = zv
            return carry
        lax.fori_loop(0, TT, zb, 0)
        pltpu.sync_copy(ob0, zshared)

    plsc.subcore_barrier()

    pltpu.sync_copy(seg_hbm.at[pl.ds(b * L, L)], seg_v.at[pl.ds(0, L)])
    seg_v[pl.ds(L, 16)] = jnp.full((16,), L, jnp.int32)
    r_start = _lower_bound(seg_v, t_base)
    r_end = _lower_bound(seg_v, t_base + TOK_PER_W)

    a0w = lax.bitwise_and(r_start, -8)
    nchw = (r_end - a0w + RC - 1) // RC

    def in_dma(ci, buf, sem):
        start = a0w + ci * RC
        d = jnp.minimum(start, L - RC)
        d = pl.multiple_of(d, 8)
        return pltpu.make_async_copy(x_hbm.at[b, pl.ds(d, RC), :], buf, sem)

    def out_dma(k, buf, sem):
        t_lo = pl.multiple_of(t_base + k * TT, TT)
        return pltpu.make_async_copy(buf, out_hbm.at[b, pl.ds(t_lo, TT), :], sem)

    @pl.when(nchw > 0)
    def _():
        in_dma(0, inbufs[0], sems_i[0]).start()

    pltpu.make_async_copy(zshared, outbufs[0], sem_z).start()

    def run_tile(kq, q, r0, lw):
        k = 2 * kq + q
        t_lo = t_base + k * TT
        r1 = _lower_bound(seg_v, t_lo + TT)
        outbuf = outbufs[q]

        pltpu.make_async_copy(zshared, outbuf, sem_z).wait()

        ci_lo = (r0 - a0w) // RC
        ci_hi = (r1 - a0w + RC - 1) // RC

        def chunk_pair(cj, carry):
            lw = carry[0]
            for p in (0, 1):
                ci = 2 * cj + p
                start = a0w + ci * RC
                d = jnp.minimum(start, L - RC)
                lo_i = jnp.maximum(r0, start)
                cnt = jnp.clip(jnp.minimum(r1, start + RC) - lo_i, 0, RC)
                fresh = (ci > lw) & (cnt > 0)

                @pl.when(fresh)
                def _():
                    in_dma(ci, inbufs[p], sems_i[p]).wait()

                    @pl.when(ci + 1 < nchw)
                    def _():
                        in_dma(ci + 1, inbufs[1 - p], sems_i[1 - p]).start()

                lw = jnp.where(fresh, ci, lw)
                inbuf = inbufs[p]

                def do_row(pr, rc):
                    prev_s = rc[0]
                    acc = rc[1:]
                    sgid = _sget(seg_v, pr)
                    lr = pr - d
                    same = sgid == prev_s
                    new_acc = []
                    for h in range(HV):
                        xv = inbuf[lr, pl.ds(16 * h, 16)]
                        new_acc.append(xv + jnp.where(same, acc[h], zv))
                    tl = sgid - t_lo
                    for h in range(HV):
                        outbuf[tl, pl.ds(16 * h, 16)] = new_acc[h]
                    return (sgid,) + tuple(new_acc)

                def row_body(i, rc):
                    return do_row(lo_i + i, rc)

                rc = lax.fori_loop(0, cnt, row_body, carry[1:])
                carry = (lw,) + rc
            return carry

        init = (lw, t_lo - 1) + tuple(zv for _ in range(HV))
        fin = lax.fori_loop(ci_lo // 2, (ci_hi + 1) // 2, chunk_pair, init)

        @pl.when(k >= 1)
        def _():
            out_dma(k - 1, outbufs[1 - q], sems_o[1 - q]).wait()

        pltpu.make_async_copy(zshared, outbufs[1 - q], sem_z).start()
        out_dma(k, outbuf, sems_o[q]).start()
        return r1, fin[0]

    def tile_pair(kq, carry):
        r0, lw = carry
        r0, lw = run_tile(kq, 0, r0, lw)
        r0, lw = run_tile(kq, 1, r0, lw)
        return r0, lw

    lax.fori_loop(0, NT // 2, tile_pair, (r_start, jnp.int32(-1)))
    pltpu.make_async_copy(zshared, outbufs[0], sem_z).wait()
    out_dma(NT - 1, outbufs[1], sems_o[1]).wait()


@jax.jit
def kernel(sequence_output, wp_segment_ids):
    seg = wp_segment_ids.astype(jnp.int32).reshape(B * L)
    run = pl.kernel(
        _body,
        out_type=jax.ShapeDtypeStruct((B, L, H), jnp.float32),
        mesh=plsc.VectorSubcoreMesh(core_axis_name="c", subcore_axis_name="s"),
        compiler_params=pltpu.CompilerParams(use_tc_tiling_on_sc=True),
        scratch_types=[
            pltpu.VMEM((L + 16,), jnp.int32),
            pltpu.VMEM((RC, H), jnp.float32),
            pltpu.VMEM((RC, H), jnp.float32),
            pltpu.VMEM((TT, H), jnp.float32),
            pltpu.VMEM((TT, H), jnp.float32),
            pltpu.VMEM_SHARED((TT, H), jnp.float32),
            pltpu.SemaphoreType.DMA,
            pltpu.SemaphoreType.DMA,
            pltpu.SemaphoreType.DMA,
            pltpu.SemaphoreType.DMA,
            pltpu.SemaphoreType.DMA,
        ],
    )
    return run(sequence_output, seg)
